# baseline (speedup 1.0000x reference)
import functools

import jax
import jax.numpy as jnp
from jax import lax
from jax.experimental import pallas as pl
from jax.experimental.pallas import tpu as pltpu
from jax.experimental.pallas import tpu_sc as plsc

DIM = 64
PAD_IDX = 0
LANES = 16
NUM_CORES = 2
NUM_SUBCORES = 16
NW = NUM_CORES * NUM_SUBCORES
CHUNK = 512


def _emb_body(xf_hbm, table_hbm, out_hbm, idx_v, rows_v, sem):
    b = xf_hbm.shape[0]
    b_per_w = b // NW
    nchunks = b_per_w // CHUNK
    wid = lax.axis_index("s") * NUM_CORES + lax.axis_index("c")
    base = wid * b_per_w

    def chunk_body(ci, _):
        off = base + ci * CHUNK
        pltpu.sync_copy(xf_hbm.at[pl.ds(off, CHUNK)], idx_v)
        pltpu.async_copy(table_hbm.at[idx_v], rows_v, sem).wait()

        def fix_group(g, _):
            iv = idx_v[pl.ds(g * LANES, LANES)]
            m = iv == PAD_IDX

            @pl.when(jnp.any(m))
            def _zero_rows():
                rowpos = g * LANES + lax.iota(jnp.int32, LANES)
                z = jnp.zeros((LANES,), jnp.float32)
                for c in range(DIM):
                    col = jnp.full((LANES,), c, jnp.int32)
                    plsc.store_scatter(rows_v, [rowpos, col], z, mask=m)

            return 0

        lax.fori_loop(0, CHUNK // LANES, fix_group, 0)
        pltpu.sync_copy(rows_v, out_hbm.at[pl.ds(off, CHUNK)])
        return 0

    lax.fori_loop(0, nchunks, chunk_body, 0)


def _make_emb(b):
    mesh = plsc.VectorSubcoreMesh(core_axis_name="c", subcore_axis_name="s")
    return functools.partial(
        pl.kernel,
        mesh=mesh,
        out_type=jax.ShapeDtypeStruct((b, DIM), jnp.float32),
        scratch_types=[
            pltpu.VMEM((CHUNK,), jnp.int32),
            pltpu.VMEM((CHUNK, DIM), jnp.float32),
            pltpu.SemaphoreType.DMA,
        ],
        compiler_params=pltpu.CompilerParams(
            needs_layout_passes=False, use_tc_tiling_on_sc=False
        ),
    )(_emb_body)


def kernel(X, table):
    xf = X.reshape(-1).astype(jnp.int32)
    out = _make_emb(xf.shape[0])(xf, table)
    return out.reshape(X.shape + (DIM,))
